# baseline (speedup 1.0000x reference)
import functools

import jax
import jax.numpy as jnp
from jax import lax
from jax.experimental import pallas as pl
from jax.experimental.pallas import tpu as pltpu
from jax.experimental.pallas import tpu_sc as plsc

N = 10000
E = 320000
H = 128
EPS = 1e-5
F32 = jnp.float32

NC, NS = 2, 16
NW = NC * NS
C = 80
K = 5
RSTR = N // NS
HH = H // NC

STRIPES = (115200, 102400, 102400)

_SCALE = 1.0507009873554805
_ALPHA = 1.6732632423543772


def _selu(x):
    return _SCALE * jnp.where(x > 0, x, _ALPHA * (jnp.exp(x) - 1.0))


def _dot(x, w):
    return jnp.dot(x, w, preferred_element_type=F32)


def _t_in_body(x_ref, w1, b1, w2, b2, wa, wb, h_ref, a_ref, b_ref):
    t = _selu(_dot(x_ref[...], w1[...]) + b1[...])
    h = _selu(_dot(t, w2[...]) + b2[...])
    h_ref[...] = h
    a_ref[...] = _dot(h, wa[...])
    b_ref[...] = _dot(h, wb[...])


def _t_in(x, w1, b1, w2, b2, wa, wb):
    blk = 2000
    g = N // blk
    full = lambda shape: pl.BlockSpec(shape, lambda i: (0, 0))
    return pl.pallas_call(
        _t_in_body,
        grid=(g,),
        in_specs=[
            pl.BlockSpec((blk, H), lambda i: (i, 0)),
            full(w1.shape), full(b1.shape), full(w2.shape), full(b2.shape),
            full(wa.shape), full(wb.shape),
        ],
        out_specs=[
            pl.BlockSpec((blk, H), lambda i: (i, 0)),
            pl.BlockSpec((blk, 64), lambda i: (i, 0)),
            pl.BlockSpec((blk, 64), lambda i: (i, 0)),
        ],
        out_shape=[
            jax.ShapeDtypeStruct((N, H), F32),
            jax.ShapeDtypeStruct((N, 64), F32),
            jax.ShapeDtypeStruct((N, 64), F32),
        ],
    )(x, w1, b1, w2, b2, wa, wb)


def _t_edge_body(gab_ref, b1, w2, b2, m_ref):
    gab = gab_ref[...].astype(F32)
    u = _selu(gab[:, :64] + gab[:, 64:] + b1[...])
    m_ref[...] = _selu(_dot(u, w2[...]) + b2[...])


def _t_edge(gab, b1, w2, b2):
    es = gab.shape[0]
    blk = 4000
    g = es // blk
    full = lambda shape: pl.BlockSpec(shape, lambda i: (0, 0))
    return pl.pallas_call(
        _t_edge_body,
        grid=(g,),
        in_specs=[
            pl.BlockSpec((blk, H), lambda i: (i, 0)),
            full(b1.shape), full(w2.shape), full(b2.shape),
        ],
        out_specs=pl.BlockSpec((blk, H), lambda i: (i, 0)),
        out_shape=jax.ShapeDtypeStruct((es, H), F32),
    )(gab, b1, w2, b2)


def _bn_selu(t, g, be):
    mu = jnp.mean(t, axis=0, keepdims=True)
    var = jnp.mean((t - mu) ** 2, axis=0, keepdims=True)
    return _selu((t - mu) * lax.rsqrt(var + EPS) * g + be)


def _t_update_body(a0, a1, a2, h_ref, ow, ob, g, be, wa, wb,
                   h1_ref, a_ref, b_ref):
    agg = a0[...] + a1[...] + a2[...]
    t = _dot(agg, ow[...]) + ob[...] + h_ref[...]
    h1 = _bn_selu(t, g[...], be[...])
    h1_ref[...] = h1
    a_ref[...] = _dot(h1, wa[...])
    b_ref[...] = _dot(h1, wb[...])


def _t_update(aggs, h, ow, ob, g, be, wa, wb):
    return pl.pallas_call(
        _t_update_body,
        out_shape=[
            jax.ShapeDtypeStruct((N, H), F32),
            jax.ShapeDtypeStruct((N, 64), F32),
            jax.ShapeDtypeStruct((N, 64), F32),
        ],
    )(*aggs, h, ow, ob, g, be, wa, wb)


def _t_final_body(a0, a1, a2, h0_ref, h1_ref, ow, ob, g, be,
                  w1a, w1b, w1c, pb1, pw2, pb2, out_ref):
    agg = a0[...] + a1[...] + a2[...]
    t = _dot(agg, ow[...]) + ob[...] + h1_ref[...]
    h2 = _bn_selu(t, g[...], be[...])
    z = (_dot(h0_ref[...], w1a[...]) + _dot(h1_ref[...], w1b[...])
         + _dot(h2, w1c[...]) + pb1[...])
    z = _selu(z)
    lo = _dot(z, pw2[...]) + pb2[...]
    m = jnp.max(lo, axis=-1, keepdims=True)
    lse = m + jnp.log(jnp.sum(jnp.exp(lo - m), axis=-1, keepdims=True))
    out_ref[...] = lo - lse


def _t_final(aggs, h0, h1, ow, ob, g, be, w1a, w1b, w1c, pb1, pw2, pb2):
    return pl.pallas_call(
        _t_final_body,
        out_shape=jax.ShapeDtypeStruct((N, 2), F32),
    )(*aggs, h0, h1, ow, ob, g, be, w1a, w1b, w1c, pb1, pw2, pb2)


_MESH = plsc.VectorSubcoreMesh(core_axis_name="c", subcore_axis_name="s",
                               num_cores=NC, num_subcores=NS)
_SC_PARAMS = pltpu.CompilerParams(use_tc_tiling_on_sc=False,
                                  disable_bounds_checks=True,
                                  disable_semaphore_checks=True)


def _drain(src_like, dst_like, sem):
    pltpu.make_async_copy(src_like, dst_like, sem).wait()


def _make_gather(es):
    epw = es // NW
    rpw = epw // C
    sch = rpw // K

    @functools.partial(
        pl.kernel,
        out_type=jax.ShapeDtypeStruct((es, H), F32),
        mesh=_MESH,
        compiler_params=_SC_PARAMS,
        scratch_types=[
            pltpu.VMEM((rpw, C), jnp.int32),
            pltpu.VMEM((rpw, C), jnp.int32),
            pltpu.VMEM((2, K * C, 64), F32),
            pltpu.VMEM((2, K * C, 64), F32),
            pltpu.SemaphoreType.DMA,
            pltpu.SemaphoreType.DMA,
            pltpu.SemaphoreType.DMA,
            pltpu.SemaphoreType.DMA,
        ],
    )
    def gather(a_hbm, b_hbm, src_hbm, dst_hbm, gab_hbm,
               idxa, idxb, bufa, bufb, gsem0, gsem1, wsem0, wsem1):
        c = lax.axis_index("c")
        s = lax.axis_index("s")
        wid = s * NC + c
        row0 = wid * rpw
        e0 = wid * epw

        pltpu.sync_copy(src_hbm.at[pl.ds(row0, rpw)], idxa)
        pltpu.sync_copy(dst_hbm.at[pl.ds(row0, rpw)], idxb)

        def fire(j, slot, gsem):
            for k in range(K):
                pltpu.async_copy(a_hbm.at[idxa.at[j * K + k]],
                                 bufa.at[slot, pl.ds(k * C, C)], gsem)
                pltpu.async_copy(b_hbm.at[idxb.at[j * K + k]],
                                 bufb.at[slot, pl.ds(k * C, C)], gsem)

        def step(j, slot, gsem, wsem, oslot, wsem_o):
            @pl.when(j + 1 < sch)
            def _():
                @pl.when(j >= 1)
                def _():
                    _drain(bufa.at[oslot],
                           gab_hbm.at[pl.ds(e0, K * C), pl.ds(0, 64)],
                           wsem_o)
                    _drain(bufb.at[oslot],
                           gab_hbm.at[pl.ds(e0, K * C), pl.ds(64, 64)],
                           wsem_o)
                fire(j + 1, oslot, gsem0 if oslot == 0 else gsem1)
            _drain(a_hbm.at[pl.ds(0, K * C)], bufa.at[slot], gsem)
            _drain(b_hbm.at[pl.ds(0, K * C)], bufb.at[slot], gsem)
            pltpu.async_copy(
                bufa.at[slot],
                gab_hbm.at[pl.ds(e0 + j * K * C, K * C), pl.ds(0, 64)], wsem)
            pltpu.async_copy(
                bufb.at[slot],
                gab_hbm.at[pl.ds(e0 + j * K * C, K * C), pl.ds(64, 64)], wsem)

        fire(0, 0, gsem0)

        def body(j, carry):
            @pl.when(j % 2 == 0)
            def _():
                step(j, 0, gsem0, wsem0, 1, wsem1)

            @pl.when(j % 2 == 1)
            def _():
                step(j, 1, gsem1, wsem1, 0, wsem0)
            return carry

        lax.fori_loop(0, sch, body, 0)
        _drain(bufa.at[0], gab_hbm.at[pl.ds(e0, K * C), pl.ds(0, 64)], wsem0)
        _drain(bufb.at[0], gab_hbm.at[pl.ds(e0, K * C), pl.ds(64, 64)], wsem0)
        _drain(bufa.at[1], gab_hbm.at[pl.ds(e0, K * C), pl.ds(0, 64)], wsem1)
        _drain(bufb.at[1], gab_hbm.at[pl.ds(e0, K * C), pl.ds(64, 64)], wsem1)

    return gather


def _make_scatter(es):
    ept = es // NS
    rpt = ept // C
    schs = rpt // K

    @functools.partial(
        pl.kernel,
        out_type=jax.ShapeDtypeStruct((N, H), F32),
        mesh=_MESH,
        compiler_params=_SC_PARAMS,
        scratch_types=[
            pltpu.VMEM((rpt, C), jnp.int32),
            pltpu.VMEM((2, K * C, HH), F32),
            pltpu.VMEM_SHARED((N, HH), F32),
            pltpu.SemaphoreType.DMA,
            pltpu.SemaphoreType.DMA,
        ],
    )
    def scatter(m_hbm, dst_hbm, z_hbm, out_hbm, idxd, mbuf, acc,
                msem0, msem1):
        c = lax.axis_index("c")
        s = lax.axis_index("s")
        row0 = s * rpt
        e0 = s * ept
        col0 = c * HH

        pltpu.sync_copy(z_hbm.at[pl.ds(s * RSTR, RSTR)],
                        acc.at[pl.ds(s * RSTR, RSTR)])
        pltpu.sync_copy(dst_hbm.at[pl.ds(row0, rpt)], idxd)
        plsc.subcore_barrier()

        def fire(j, slot, msem):
            pltpu.async_copy(
                m_hbm.at[pl.ds(e0 + j * K * C, K * C), pl.ds(col0, HH)],
                mbuf.at[slot], msem)

        def step(j, slot, msem, oslot, msem_o):
            @pl.when(j + 1 < schs)
            def _():
                fire(j + 1, oslot, msem_o)
            _drain(m_hbm.at[pl.ds(0, K * C), pl.ds(col0, HH)],
                   mbuf.at[slot], msem)
            for k in range(K):
                pltpu.sync_copy(mbuf.at[slot, pl.ds(k * C, C)],
                                acc.at[idxd.at[j * K + k]], add=True)

        fire(0, 0, msem0)

        def body(j, carry):
            @pl.when(j % 2 == 0)
            def _():
                step(j, 0, msem0, 1, msem1)

            @pl.when(j % 2 == 1)
            def _():
                step(j, 1, msem1, 0, msem0)
            return carry

        lax.fori_loop(0, schs, body, 0)

        plsc.subcore_barrier()
        pltpu.sync_copy(acc.at[pl.ds(s * RSTR, RSTR)],
                        out_hbm.at[pl.ds(s * RSTR, RSTR), pl.ds(col0, HH)])

    return scatter


_GATHERS = {es: _make_gather(es) for es in set(STRIPES)}
_SCATTERS = {es: _make_scatter(es) for es in set(STRIPES)}


def kernel(x, edge_index, pin_w1, pin_b1, pin_w2, pin_b2,
           l0_ew1, l0_eb1, l0_ew2, l0_eb2, l0_ow, l0_ob, l0_g, l0_be,
           l1_ew1, l1_eb1, l1_ew2, l1_eb2, l1_ow, l1_ob, l1_g, l1_be,
           pout_w1, pout_b1, pout_w2, pout_b2):
    srcs, dsts = [], []
    off = 0
    for es in STRIPES:
        srcs.append(edge_index[0, off:off + es].reshape(es // C, C))
        dsts.append(edge_index[1, off:off + es].reshape(es // C, C))
        off += es
    zeros = jnp.zeros((N, HH), F32)

    r2 = lambda v: v.reshape(1, -1)

    def layer(a, b, eb1, ew2, eb2):
        gabs = [_GATHERS[es](a, b, srcs[t], dsts[t])
                for t, es in enumerate(STRIPES)]
        ms = [_t_edge(gabs[t], r2(eb1), ew2, r2(eb2))
              for t in range(len(STRIPES))]
        return [_SCATTERS[es](ms[t], dsts[t], zeros)
                for t, es in enumerate(STRIPES)]

    h0, a, b = _t_in(x, pin_w1, r2(pin_b1), pin_w2, r2(pin_b2),
                     l0_ew1[:H], l0_ew1[H:])
    aggs = layer(a, b, l0_eb1, l0_ew2, l0_eb2)
    h1, a, b = _t_update(aggs, h0, l0_ow, r2(l0_ob), r2(l0_g), r2(l0_be),
                         l1_ew1[:H], l1_ew1[H:])
    aggs = layer(a, b, l1_eb1, l1_ew2, l1_eb2)
    return _t_final(aggs, h0, h1, l1_ow, r2(l1_ob), r2(l1_g), r2(l1_be),
                    pout_w1[:H], pout_w1[H:2 * H], pout_w1[2 * H:],
                    r2(pout_b1), pout_w2, r2(pout_b2))


# speedup vs baseline: 1.0135x; 1.0135x over previous
import functools

import jax
import jax.numpy as jnp
from jax import lax
from jax.experimental import pallas as pl
from jax.experimental.pallas import tpu as pltpu
from jax.experimental.pallas import tpu_sc as plsc

N = 10000
E = 320000
H = 128
EPS = 1e-5
F32 = jnp.float32

NC, NS = 2, 16
NW = NC * NS
C = 80
K = 5
RSTR = N // NS
HH = H // NC

STRIPES = (115200, 102400, 102400)

_SCALE = 1.0507009873554805
_ALPHA = 1.6732632423543772


def _selu(x):
    return _SCALE * jnp.where(x > 0, x, _ALPHA * (jnp.exp(x) - 1.0))


def _dot(x, w):
    return jnp.dot(x, w, preferred_element_type=F32)


def _t_in_body(x_ref, w1, b1, w2, b2, wa, wb, h_ref, a_ref, b_ref):
    t = _selu(_dot(x_ref[...], w1[...]) + b1[...])
    h = _selu(_dot(t, w2[...]) + b2[...])
    h_ref[...] = h
    a_ref[...] = _dot(h, wa[...])
    b_ref[...] = _dot(h, wb[...])


def _t_in(x, w1, b1, w2, b2, wa, wb):
    blk = 2000
    g = N // blk
    full = lambda shape: pl.BlockSpec(shape, lambda i: (0, 0))
    return pl.pallas_call(
        _t_in_body,
        grid=(g,),
        in_specs=[
            pl.BlockSpec((blk, H), lambda i: (i, 0)),
            full(w1.shape), full(b1.shape), full(w2.shape), full(b2.shape),
            full(wa.shape), full(wb.shape),
        ],
        out_specs=[
            pl.BlockSpec((blk, H), lambda i: (i, 0)),
            pl.BlockSpec((blk, 64), lambda i: (i, 0)),
            pl.BlockSpec((blk, 64), lambda i: (i, 0)),
        ],
        out_shape=[
            jax.ShapeDtypeStruct((N, H), F32),
            jax.ShapeDtypeStruct((N, 64), F32),
            jax.ShapeDtypeStruct((N, 64), F32),
        ],
    )(x, w1, b1, w2, b2, wa, wb)


def _t_edge_body(gab_ref, b1, w2, b2, m_ref):
    gab = gab_ref[...].astype(F32)
    u = _selu(gab[:, :64] + gab[:, 64:] + b1[...])
    m_ref[...] = _selu(_dot(u, w2[...]) + b2[...])


def _t_edge(gab, b1, w2, b2):
    es = gab.shape[0]
    blk = 3200
    g = es // blk
    assert es % blk == 0
    full = lambda shape: pl.BlockSpec(shape, lambda i: (0, 0))
    return pl.pallas_call(
        _t_edge_body,
        grid=(g,),
        in_specs=[
            pl.BlockSpec((blk, H), lambda i: (i, 0)),
            full(b1.shape), full(w2.shape), full(b2.shape),
        ],
        out_specs=pl.BlockSpec((blk, H), lambda i: (i, 0)),
        out_shape=jax.ShapeDtypeStruct((es, H), F32),
    )(gab, b1, w2, b2)


def _bn_selu(t, g, be):
    mu = jnp.mean(t, axis=0, keepdims=True)
    var = jnp.mean((t - mu) ** 2, axis=0, keepdims=True)
    return _selu((t - mu) * lax.rsqrt(var + EPS) * g + be)


def _t_update_body(a0, a1, a2, h_ref, ow, ob, g, be, wa, wb,
                   h1_ref, a_ref, b_ref):
    agg = a0[...] + a1[...] + a2[...]
    t = _dot(agg, ow[...]) + ob[...] + h_ref[...]
    h1 = _bn_selu(t, g[...], be[...])
    h1_ref[...] = h1
    a_ref[...] = _dot(h1, wa[...])
    b_ref[...] = _dot(h1, wb[...])


def _t_update(aggs, h, ow, ob, g, be, wa, wb):
    return pl.pallas_call(
        _t_update_body,
        out_shape=[
            jax.ShapeDtypeStruct((N, H), F32),
            jax.ShapeDtypeStruct((N, 64), F32),
            jax.ShapeDtypeStruct((N, 64), F32),
        ],
    )(*aggs, h, ow, ob, g, be, wa, wb)


def _t_final_body(a0, a1, a2, h0_ref, h1_ref, ow, ob, g, be,
                  w1a, w1b, w1c, pb1, pw2, pb2, out_ref):
    agg = a0[...] + a1[...] + a2[...]
    t = _dot(agg, ow[...]) + ob[...] + h1_ref[...]
    h2 = _bn_selu(t, g[...], be[...])
    z = (_dot(h0_ref[...], w1a[...]) + _dot(h1_ref[...], w1b[...])
         + _dot(h2, w1c[...]) + pb1[...])
    z = _selu(z)
    lo = _dot(z, pw2[...]) + pb2[...]
    m = jnp.max(lo, axis=-1, keepdims=True)
    lse = m + jnp.log(jnp.sum(jnp.exp(lo - m), axis=-1, keepdims=True))
    out_ref[...] = lo - lse


def _t_final(aggs, h0, h1, ow, ob, g, be, w1a, w1b, w1c, pb1, pw2, pb2):
    return pl.pallas_call(
        _t_final_body,
        out_shape=jax.ShapeDtypeStruct((N, 2), F32),
    )(*aggs, h0, h1, ow, ob, g, be, w1a, w1b, w1c, pb1, pw2, pb2)


_MESH = plsc.VectorSubcoreMesh(core_axis_name="c", subcore_axis_name="s",
                               num_cores=NC, num_subcores=NS)
_SC_PARAMS = pltpu.CompilerParams(use_tc_tiling_on_sc=False)


def _drain(src_like, dst_like, sem):
    pltpu.make_async_copy(src_like, dst_like, sem).wait()


def _make_gather(es):
    epw = es // NW
    rpw = epw // C
    sch = rpw // K

    @functools.partial(
        pl.kernel,
        out_type=jax.ShapeDtypeStruct((es, H), F32),
        mesh=_MESH,
        compiler_params=_SC_PARAMS,
        scratch_types=[
            pltpu.VMEM((rpw, C), jnp.int32),
            pltpu.VMEM((rpw, C), jnp.int32),
            pltpu.VMEM((2, K * C, 64), F32),
            pltpu.VMEM((2, K * C, 64), F32),
            pltpu.SemaphoreType.DMA,
            pltpu.SemaphoreType.DMA,
            pltpu.SemaphoreType.DMA,
            pltpu.SemaphoreType.DMA,
        ],
    )
    def gather(a_hbm, b_hbm, src_hbm, dst_hbm, gab_hbm,
               idxa, idxb, bufa, bufb, gsem0, gsem1, wsem0, wsem1):
        c = lax.axis_index("c")
        s = lax.axis_index("s")
        wid = s * NC + c
        row0 = wid * rpw
        e0 = wid * epw

        pltpu.sync_copy(src_hbm.at[pl.ds(row0, rpw)], idxa)
        pltpu.sync_copy(dst_hbm.at[pl.ds(row0, rpw)], idxb)

        def fire(j, slot, gsem):
            for k in range(K):
                pltpu.async_copy(a_hbm.at[idxa.at[j * K + k]],
                                 bufa.at[slot, pl.ds(k * C, C)], gsem)
                pltpu.async_copy(b_hbm.at[idxb.at[j * K + k]],
                                 bufb.at[slot, pl.ds(k * C, C)], gsem)

        def step(j, slot, gsem, wsem, oslot, wsem_o):
            @pl.when(j + 1 < sch)
            def _():
                @pl.when(j >= 1)
                def _():
                    _drain(bufa.at[oslot],
                           gab_hbm.at[pl.ds(e0, K * C), pl.ds(0, 64)],
                           wsem_o)
                    _drain(bufb.at[oslot],
                           gab_hbm.at[pl.ds(e0, K * C), pl.ds(64, 64)],
                           wsem_o)
                fire(j + 1, oslot, gsem0 if oslot == 0 else gsem1)
            _drain(a_hbm.at[pl.ds(0, K * C)], bufa.at[slot], gsem)
            _drain(b_hbm.at[pl.ds(0, K * C)], bufb.at[slot], gsem)
            pltpu.async_copy(
                bufa.at[slot],
                gab_hbm.at[pl.ds(e0 + j * K * C, K * C), pl.ds(0, 64)], wsem)
            pltpu.async_copy(
                bufb.at[slot],
                gab_hbm.at[pl.ds(e0 + j * K * C, K * C), pl.ds(64, 64)], wsem)

        fire(0, 0, gsem0)

        def body(j, carry):
            @pl.when(j % 2 == 0)
            def _():
                step(j, 0, gsem0, wsem0, 1, wsem1)

            @pl.when(j % 2 == 1)
            def _():
                step(j, 1, gsem1, wsem1, 0, wsem0)
            return carry

        lax.fori_loop(0, sch, body, 0)
        _drain(bufa.at[0], gab_hbm.at[pl.ds(e0, K * C), pl.ds(0, 64)], wsem0)
        _drain(bufb.at[0], gab_hbm.at[pl.ds(e0, K * C), pl.ds(64, 64)], wsem0)
        _drain(bufa.at[1], gab_hbm.at[pl.ds(e0, K * C), pl.ds(0, 64)], wsem1)
        _drain(bufb.at[1], gab_hbm.at[pl.ds(e0, K * C), pl.ds(64, 64)], wsem1)

    return gather


def _make_scatter(es):
    ept = es // NS
    rpt = ept // C
    schs = rpt // K

    @functools.partial(
        pl.kernel,
        out_type=jax.ShapeDtypeStruct((N, H), F32),
        mesh=_MESH,
        compiler_params=_SC_PARAMS,
        scratch_types=[
            pltpu.VMEM((rpt, C), jnp.int32),
            pltpu.VMEM((2, K * C, HH), F32),
            pltpu.VMEM_SHARED((N, HH), F32),
            pltpu.SemaphoreType.DMA,
            pltpu.SemaphoreType.DMA,
        ],
    )
    def scatter(m_hbm, dst_hbm, z_hbm, out_hbm, idxd, mbuf, acc,
                msem0, msem1):
        c = lax.axis_index("c")
        s = lax.axis_index("s")
        row0 = s * rpt
        e0 = s * ept
        col0 = c * HH

        pltpu.sync_copy(z_hbm.at[pl.ds(s * RSTR, RSTR)],
                        acc.at[pl.ds(s * RSTR, RSTR)])
        pltpu.sync_copy(dst_hbm.at[pl.ds(row0, rpt)], idxd)
        plsc.subcore_barrier()

        def fire(j, slot, msem):
            pltpu.async_copy(
                m_hbm.at[pl.ds(e0 + j * K * C, K * C), pl.ds(col0, HH)],
                mbuf.at[slot], msem)

        def step(j, slot, msem, oslot, msem_o):
            @pl.when(j + 1 < schs)
            def _():
                fire(j + 1, oslot, msem_o)
            _drain(m_hbm.at[pl.ds(0, K * C), pl.ds(col0, HH)],
                   mbuf.at[slot], msem)
            for k in range(K):
                pltpu.sync_copy(mbuf.at[slot, pl.ds(k * C, C)],
                                acc.at[idxd.at[j * K + k]], add=True)

        fire(0, 0, msem0)

        def body(j, carry):
            @pl.when(j % 2 == 0)
            def _():
                step(j, 0, msem0, 1, msem1)

            @pl.when(j % 2 == 1)
            def _():
                step(j, 1, msem1, 0, msem0)
            return carry

        lax.fori_loop(0, schs, body, 0)

        plsc.subcore_barrier()
        pltpu.sync_copy(acc.at[pl.ds(s * RSTR, RSTR)],
                        out_hbm.at[pl.ds(s * RSTR, RSTR), pl.ds(col0, HH)])

    return scatter


_GATHERS = {es: _make_gather(es) for es in set(STRIPES)}
_SCATTERS = {es: _make_scatter(es) for es in set(STRIPES)}


def kernel(x, edge_index, pin_w1, pin_b1, pin_w2, pin_b2,
           l0_ew1, l0_eb1, l0_ew2, l0_eb2, l0_ow, l0_ob, l0_g, l0_be,
           l1_ew1, l1_eb1, l1_ew2, l1_eb2, l1_ow, l1_ob, l1_g, l1_be,
           pout_w1, pout_b1, pout_w2, pout_b2):
    srcs, dsts = [], []
    off = 0
    for es in STRIPES:
        srcs.append(edge_index[0, off:off + es].reshape(es // C, C))
        dsts.append(edge_index[1, off:off + es].reshape(es // C, C))
        off += es
    zeros = jnp.zeros((N, HH), F32)

    r2 = lambda v: v.reshape(1, -1)

    def layer(a, b, eb1, ew2, eb2):
        gabs = [_GATHERS[es](a, b, srcs[t], dsts[t])
                for t, es in enumerate(STRIPES)]
        ms = [_t_edge(gabs[t], r2(eb1), ew2, r2(eb2))
              for t in range(len(STRIPES))]
        return [_SCATTERS[es](ms[t], dsts[t], zeros)
                for t, es in enumerate(STRIPES)]

    h0, a, b = _t_in(x, pin_w1, r2(pin_b1), pin_w2, r2(pin_b2),
                     l0_ew1[:H], l0_ew1[H:])
    aggs = layer(a, b, l0_eb1, l0_ew2, l0_eb2)
    h1, a, b = _t_update(aggs, h0, l0_ow, r2(l0_ob), r2(l0_g), r2(l0_be),
                         l1_ew1[:H], l1_ew1[H:])
    aggs = layer(a, b, l1_eb1, l1_ew2, l1_eb2)
    return _t_final(aggs, h0, h1, l1_ow, r2(l1_ob), r2(l1_g), r2(l1_be),
                    pout_w1[:H], pout_w1[H:2 * H], pout_w1[2 * H:],
                    r2(pout_b1), pout_w2, r2(pout_b2))
